# baseline (speedup 1.0000x reference)
import functools

import jax
import jax.numpy as jnp
from jax import lax
from jax.experimental import pallas as pl
from jax.experimental.pallas import tpu as pltpu
from jax.experimental.pallas import tpu_sc as plsc

DIM = 32
VOCAB = 1000000
NUM_CORES = 2
NUM_SUBCORES = 16
NUM_WORKERS = NUM_CORES * NUM_SUBCORES
NTC = 7813
CHUNK = 128
NBUF = 8


@jax.jit
def _transpose_sc(tabT, tail16):
    mesh = plsc.VectorSubcoreMesh(core_axis_name="c", subcore_axis_name="s")

    @functools.partial(
        pl.kernel,
        out_type=jax.ShapeDtypeStruct((VOCAB * DIM // 128, 128), jnp.float32),
        mesh=mesh,
        scratch_types=[
            pltpu.VMEM((32, 128), jnp.float32),
            pltpu.VMEM((32, 128), jnp.float32),
            pltpu.SemaphoreType.DMA,
        ],
        compiler_params=pltpu.CompilerParams(
            use_tc_tiling_on_sc=True, needs_layout_passes=False
        ),
    )
    def k(tabT_hbm, tail16_hbm, out_hbm, in_v, out_v, sem):
        wid = lax.axis_index("s") * NUM_CORES + lax.axis_index("c")
        iota16 = lax.iota(jnp.int32, 16)

        def shuffle(nro):
            for ro in range(nro):
                for c in range(8):
                    row_idx = iota16 + 16 * (c % 2)
                    col_idx = jnp.full((16,), 4 * ro + c // 2, jnp.int32)
                    out_v[ro, pl.ds(16 * c, 16)] = plsc.load_gather(
                        in_v, [row_idx, col_idx]
                    )

        def body(i, carry):
            tc = wid + i * NUM_WORKERS
            for g in range(4):
                pltpu.async_copy(
                    tabT_hbm.at[pl.ds(8 * g, 8), pl.ds(tc * 128, 128)],
                    in_v.at[pl.ds(8 * g, 8), :],
                    sem,
                )
            for g in range(4):
                pltpu.make_async_copy(
                    tabT_hbm.at[pl.ds(8 * g, 8), pl.ds(tc * 128, 128)],
                    in_v.at[pl.ds(8 * g, 8), :],
                    sem,
                ).wait()
            shuffle(32)
            pltpu.sync_copy(out_v, out_hbm.at[pl.ds(tc * 32, 32)])
            return carry

        lax.fori_loop(0, NTC // NUM_WORKERS, body, 0)

        @pl.when(wid < 4)
        def _tail_full():
            body(NTC // NUM_WORKERS, 0)

        @pl.when(wid == 4)
        def _tail_partial():
            pltpu.sync_copy(tail16_hbm, in_v.at[pl.ds(0, 16)])
            pltpu.sync_copy(
                in_v.at[pl.ds(0, 16)],
                out_hbm.at[pl.ds(VOCAB * 32 // 128 - 16, 16)],
            )

    return k(tabT, tail16)


@functools.partial(jax.jit, static_argnames=("b_per_w",))
def _gather_sc(table, idx_flat, b_per_w):
    n_chunks = b_per_w // CHUNK
    n_groups = n_chunks // NBUF
    mesh = plsc.VectorSubcoreMesh(core_axis_name="c", subcore_axis_name="s")

    @functools.partial(
        pl.kernel,
        out_type=jax.ShapeDtypeStruct((idx_flat.shape[0], DIM), jnp.float32),
        mesh=mesh,
        scratch_types=[
            pltpu.VMEM((b_per_w,), jnp.int32),
            [pltpu.VMEM((CHUNK, DIM), jnp.float32) for _ in range(NBUF)],
            [pltpu.SemaphoreType.DMA for _ in range(NBUF)],
            [pltpu.SemaphoreType.DMA for _ in range(NBUF)],
        ],
        compiler_params=pltpu.CompilerParams(use_tc_tiling_on_sc=False),
    )
    def k(table_hbm, idx_hbm, out_hbm, idx_v, rows, gsems, wsems):
        wid = lax.axis_index("s") * NUM_CORES + lax.axis_index("c")
        base = wid * b_per_w
        pltpu.sync_copy(idx_hbm.at[pl.ds(base, b_per_w)], idx_v)

        def gather_start(j, b):
            pltpu.async_copy(
                table_hbm.at[idx_v.at[pl.ds(j * CHUNK, CHUNK)]], rows[b], gsems[b]
            )

        def gather_wait(b):
            pltpu.make_async_copy(
                table_hbm.at[idx_v.at[pl.ds(0, CHUNK)]], rows[b], gsems[b]
            ).wait()

        def write_start(j, b):
            pltpu.async_copy(
                rows[b], out_hbm.at[pl.ds(base + j * CHUNK, CHUNK)], wsems[b]
            )

        def write_wait(j, b):
            pltpu.make_async_copy(
                rows[b], out_hbm.at[pl.ds(base + j * CHUNK, CHUNK)], wsems[b]
            ).wait()

        for b in range(NBUF):
            gather_start(b, b)

        def body(g, carry):
            j0 = g * NBUF
            for b in range(NBUF):
                gather_wait(b)
                write_start(j0 + b, b)
            for b in range(NBUF):
                write_wait(j0 + b, b)
                gather_start(j0 + b + NBUF, b)
            return carry

        lax.fori_loop(0, n_groups - 1, body, 0)
        j0 = (n_groups - 1) * NBUF
        for b in range(NBUF):
            gather_wait(b)
            write_start(j0 + b, b)
        for b in range(NBUF):
            write_wait(j0 + b, b)

    return k(table, idx_flat)


def kernel(table, indices):
    batch, fields = indices.shape
    total = batch * fields
    tail16 = table[VOCAB - 64 :, :].reshape(16, 128)
    rows_tab = _transpose_sc(table.T, tail16).reshape(VOCAB, DIM)
    idx_flat = indices.reshape(total)
    out = _gather_sc(rows_tab, idx_flat, total // NUM_WORKERS)
    return out.reshape(batch, fields, DIM)


# speedup vs baseline: 1.2764x; 1.2764x over previous
import functools

import jax
import jax.numpy as jnp
from jax import lax
from jax.experimental import pallas as pl
from jax.experimental.pallas import tpu as pltpu
from jax.experimental.pallas import tpu_sc as plsc

DIM = 32
VOCAB = 1000000
NUM_CORES = 2
NUM_SUBCORES = 16
NUM_WORKERS = NUM_CORES * NUM_SUBCORES
NTC = 7813
CHUNK = 128
NBUF = 8


KTC = 4
NGRP = (NTC - 1) // KTC


@jax.jit
def _transpose_sc(tabT, tail16):
    mesh = plsc.VectorSubcoreMesh(core_axis_name="c", subcore_axis_name="s")

    @functools.partial(
        pl.kernel,
        out_type=jax.ShapeDtypeStruct((VOCAB * DIM // 128, 128), jnp.float32),
        mesh=mesh,
        scratch_types=[
            [pltpu.VMEM((128, 128), jnp.float32) for _ in range(2)],
            [pltpu.VMEM((128, 128), jnp.float32) for _ in range(2)],
            [pltpu.SemaphoreType.DMA for _ in range(2)],
            [pltpu.SemaphoreType.DMA for _ in range(2)],
        ],
        compiler_params=pltpu.CompilerParams(
            use_tc_tiling_on_sc=True, needs_layout_passes=False
        ),
    )
    def k(tabT_hbm, tail16_hbm, out_hbm, in_v, out_v, rsems, wsems):
        wid = lax.axis_index("s") * NUM_CORES + lax.axis_index("c")
        iota16 = lax.iota(jnp.int32, 16)
        n_i = 61 + jnp.where(wid < NGRP - 61 * NUM_WORKERS, 1, 0)

        def grp(i):
            return wid + i * NUM_WORKERS

        def read_start(i, buf):
            tc0 = grp(i) * KTC
            for t in range(KTC):
                for g in range(4):
                    pltpu.async_copy(
                        tabT_hbm.at[pl.ds(8 * g, 8), pl.ds((tc0 + t) * 128, 128)],
                        in_v[buf].at[pl.ds(32 * t + 8 * g, 8), :],
                        rsems[buf],
                    )

        def read_wait(i, buf):
            tc0 = grp(i) * KTC
            for t in range(KTC):
                for g in range(4):
                    pltpu.make_async_copy(
                        tabT_hbm.at[pl.ds(8 * g, 8), pl.ds((tc0 + t) * 128, 128)],
                        in_v[buf].at[pl.ds(32 * t + 8 * g, 8), :],
                        rsems[buf],
                    ).wait()

        def write_start(i, buf):
            pltpu.async_copy(
                out_v[buf], out_hbm.at[pl.ds(grp(i) * 128, 128)], wsems[buf]
            )

        def write_wait(i, buf):
            pltpu.make_async_copy(
                out_v[buf], out_hbm.at[pl.ds(grp(i) * 128, 128)], wsems[buf]
            ).wait()

        def shuffle(buf):
            for t in range(KTC):
                for c in range(8):
                    row_idx = iota16 + (16 * (c % 2) + 32 * t)
                    for ro in range(32):
                        col_idx = jnp.full((16,), 4 * ro + c // 2, jnp.int32)
                        out_v[buf][32 * t + ro, pl.ds(16 * c, 16)] = (
                            plsc.load_gather(in_v[buf], [row_idx, col_idx])
                        )

        read_start(0, 0)

        def step(i, buf):
            @pl.when(i < n_i)
            def _():
                @pl.when(i + 1 < n_i)
                def _():
                    read_start(i + 1, 1 - buf)

                read_wait(i, buf)

                @pl.when(i >= 2)
                def _():
                    write_wait(i - 2, buf)

                shuffle(buf)
                write_start(i, buf)

        def body(p, carry):
            step(2 * p, 0)
            step(2 * p + 1, 1)
            return carry

        lax.fori_loop(0, 31, body, 0)

        @pl.when(n_i == 62)
        def _():
            write_wait(60, 0)
            write_wait(61, 1)

        @pl.when(n_i == 61)
        def _():
            write_wait(59, 1)
            write_wait(60, 0)

        @pl.when(wid == 4)
        def _tail_partial():
            pltpu.sync_copy(tail16_hbm, in_v[0].at[pl.ds(0, 16)])
            pltpu.sync_copy(
                in_v[0].at[pl.ds(0, 16)],
                out_hbm.at[pl.ds(VOCAB * 32 // 128 - 16, 16)],
            )

    return k(tabT, tail16)


@functools.partial(jax.jit, static_argnames=("b_per_w",))
def _gather_sc(table, idx_flat, b_per_w):
    n_chunks = b_per_w // CHUNK
    n_groups = n_chunks // NBUF
    mesh = plsc.VectorSubcoreMesh(core_axis_name="c", subcore_axis_name="s")

    @functools.partial(
        pl.kernel,
        out_type=jax.ShapeDtypeStruct((idx_flat.shape[0], DIM), jnp.float32),
        mesh=mesh,
        scratch_types=[
            pltpu.VMEM((b_per_w,), jnp.int32),
            [pltpu.VMEM((CHUNK, DIM), jnp.float32) for _ in range(NBUF)],
            [pltpu.SemaphoreType.DMA for _ in range(NBUF)],
            [pltpu.SemaphoreType.DMA for _ in range(NBUF)],
        ],
        compiler_params=pltpu.CompilerParams(use_tc_tiling_on_sc=False),
    )
    def k(table_hbm, idx_hbm, out_hbm, idx_v, rows, gsems, wsems):
        wid = lax.axis_index("s") * NUM_CORES + lax.axis_index("c")
        base = wid * b_per_w
        pltpu.sync_copy(idx_hbm.at[pl.ds(base, b_per_w)], idx_v)

        def gather_start(j, b):
            pltpu.async_copy(
                table_hbm.at[idx_v.at[pl.ds(j * CHUNK, CHUNK)]], rows[b], gsems[b]
            )

        def gather_wait(b):
            pltpu.make_async_copy(
                table_hbm.at[idx_v.at[pl.ds(0, CHUNK)]], rows[b], gsems[b]
            ).wait()

        def write_start(j, b):
            pltpu.async_copy(
                rows[b], out_hbm.at[pl.ds(base + j * CHUNK, CHUNK)], wsems[b]
            )

        def write_wait(j, b):
            pltpu.make_async_copy(
                rows[b], out_hbm.at[pl.ds(base + j * CHUNK, CHUNK)], wsems[b]
            ).wait()

        for b in range(NBUF):
            gather_start(b, b)

        def body(g, carry):
            j0 = g * NBUF
            for b in range(NBUF):
                gather_wait(b)
                write_start(j0 + b, b)
            for b in range(NBUF):
                write_wait(j0 + b, b)
                gather_start(j0 + b + NBUF, b)
            return carry

        lax.fori_loop(0, n_groups - 1, body, 0)
        j0 = (n_groups - 1) * NBUF
        for b in range(NBUF):
            gather_wait(b)
            write_start(j0 + b, b)
        for b in range(NBUF):
            write_wait(j0 + b, b)

    return k(table, idx_flat)


def kernel(table, indices):
    batch, fields = indices.shape
    total = batch * fields
    tail16 = table[VOCAB - 64 :, :].reshape(16, 128)
    rows_tab = _transpose_sc(table.T, tail16).reshape(VOCAB, DIM)
    idx_flat = indices.reshape(total)
    out = _gather_sc(rows_tab, idx_flat, total // NUM_WORKERS)
    return out.reshape(batch, fields, DIM)


# speedup vs baseline: 1.5382x; 1.2051x over previous
import functools

import jax
import jax.numpy as jnp
from jax import lax
from jax.experimental import pallas as pl
from jax.experimental.pallas import tpu as pltpu
from jax.experimental.pallas import tpu_sc as plsc

DIM = 32
VOCAB = 1000000
NUM_CORES = 2
NUM_SUBCORES = 16
NUM_WORKERS = NUM_CORES * NUM_SUBCORES
CHUNK = 128
NBUF = 4


@functools.partial(jax.jit, static_argnames=("b_per_w",))
def _gather_sc(tpad, idx_flat, b_per_w):
    n_chunks = b_per_w // CHUNK
    n_groups = n_chunks // NBUF
    mesh = plsc.VectorSubcoreMesh(core_axis_name="c", subcore_axis_name="s")

    @functools.partial(
        pl.kernel,
        out_type=jax.ShapeDtypeStruct((idx_flat.shape[0], DIM), jnp.float32),
        mesh=mesh,
        scratch_types=[
            pltpu.VMEM((b_per_w,), jnp.int32),
            [pltpu.VMEM((CHUNK, 128), jnp.float32) for _ in range(NBUF)],
            [pltpu.SemaphoreType.DMA for _ in range(NBUF)],
            [pltpu.SemaphoreType.DMA for _ in range(NBUF)],
        ],
        compiler_params=pltpu.CompilerParams(use_tc_tiling_on_sc=False),
    )
    def k(table_hbm, idx_hbm, out_hbm, idx_v, rows, gsems, wsems):
        wid = lax.axis_index("s") * NUM_CORES + lax.axis_index("c")
        base = wid * b_per_w
        pltpu.sync_copy(idx_hbm.at[pl.ds(base, b_per_w)], idx_v)

        def gather_start(j, b):
            pltpu.async_copy(
                table_hbm.at[idx_v.at[pl.ds(j * CHUNK, CHUNK)]], rows[b], gsems[b]
            )

        def gather_wait(b):
            pltpu.make_async_copy(
                table_hbm.at[idx_v.at[pl.ds(0, CHUNK)]], rows[b], gsems[b]
            ).wait()

        def write_start(j, b):
            pltpu.async_copy(
                rows[b].at[:, pl.ds(0, DIM)],
                out_hbm.at[pl.ds(base + j * CHUNK, CHUNK)],
                wsems[b],
            )

        def write_wait(j, b):
            pltpu.make_async_copy(
                rows[b].at[:, pl.ds(0, DIM)],
                out_hbm.at[pl.ds(base + j * CHUNK, CHUNK)],
                wsems[b],
            ).wait()

        for b in range(NBUF):
            gather_start(b, b)

        def body(g, carry):
            j0 = g * NBUF
            for b in range(NBUF):
                gather_wait(b)
                write_start(j0 + b, b)
            for b in range(NBUF):
                write_wait(j0 + b, b)
                gather_start(j0 + b + NBUF, b)
            return carry

        lax.fori_loop(0, n_groups - 1, body, 0)
        j0 = (n_groups - 1) * NBUF
        for b in range(NBUF):
            gather_wait(b)
            write_start(j0 + b, b)
        for b in range(NBUF):
            write_wait(j0 + b, b)

    return k(tpad, idx_flat)


def kernel(table, indices):
    batch, fields = indices.shape
    total = batch * fields
    tpad = jnp.pad(table, ((0, 0), (0, 128 - DIM)))
    idx_flat = indices.reshape(total)
    out = _gather_sc(tpad, idx_flat, total // NUM_WORKERS)
    return out.reshape(batch, fields, DIM)


# speedup vs baseline: 1.7218x; 1.1194x over previous
import functools

import jax
import jax.numpy as jnp
from jax import lax
from jax.experimental import pallas as pl
from jax.experimental.pallas import tpu as pltpu
from jax.experimental.pallas import tpu_sc as plsc

DIM = 32
VOCAB = 1000000
FIELDS = 26
NUM_CORES = 2
NUM_SUBCORES = 16
NUM_WORKERS = NUM_CORES * NUM_SUBCORES
CHUNK = 104
BLK_B = 16
CPB = BLK_B * FIELDS // CHUNK
NBLK = 512 // BLK_B
SPAD = 17


@jax.jit
def _gather_sc(tpad, idx_flat):
    b_per_w = idx_flat.shape[0] // NUM_WORKERS
    mesh = plsc.VectorSubcoreMesh(core_axis_name="c", subcore_axis_name="s")

    @functools.partial(
        pl.kernel,
        out_type=jax.ShapeDtypeStruct((FIELDS, 4, 128, 8, 128), jnp.float32),
        mesh=mesh,
        scratch_types=[
            pltpu.VMEM((13312,), jnp.int32),
            [pltpu.VMEM((CHUNK, 128), jnp.float32) for _ in range(2)],
            pltpu.VMEM((FIELDS, 32, SPAD), jnp.float32),
            [pltpu.SemaphoreType.DMA for _ in range(2)],
            pltpu.SemaphoreType.DMA,
        ],
        compiler_params=pltpu.CompilerParams(
            use_tc_tiling_on_sc=False, needs_layout_passes=False
        ),
    )
    def k(table_hbm, idx_hbm, out_hbm, idx_v, rows, stage, gsems, wsem):
        wid = lax.axis_index("s") * NUM_CORES + lax.axis_index("c")
        base = wid * b_per_w
        pltpu.sync_copy(idx_hbm.at[pl.ds(base, b_per_w)], idx_v)

        iota16 = lax.iota(jnp.int32, 16)

        def gather_start(j, b):
            pltpu.async_copy(
                table_hbm.at[idx_v.at[pl.ds(j * CHUNK, CHUNK)]], rows[b], gsems[b]
            )

        def gather_wait(b):
            pltpu.make_async_copy(
                table_hbm.at[idx_v.at[pl.ds(0, CHUNK)]], rows[b], gsems[b]
            ).wait()

        def scatter(q, b):
            for r in range(CHUNK):
                flat = q * CHUNK + r
                f = flat % FIELDS
                bi = flat // FIELDS
                f_splat = jnp.full((16,), f, jnp.int32)
                bi_splat = jnp.full((16,), bi, jnp.int32)
                for h in range(2):
                    plsc.store_scatter(
                        stage,
                        [f_splat, iota16 + 16 * h, bi_splat],
                        rows[b][r, pl.ds(16 * h, 16)],
                    )

        def write_start(f, dg, blk):
            bg = 4 * wid + lax.div(blk, 8)
            bi0 = BLK_B * lax.rem(blk, 8)
            pltpu.async_copy(
                stage.at[f, pl.ds(8 * dg, 8), pl.ds(0, BLK_B)],
                out_hbm.at[f, dg, bg, :, pl.ds(bi0, BLK_B)],
                wsem,
            )

        def write_wait(f, dg, blk):
            bg = 4 * wid + lax.div(blk, 8)
            bi0 = BLK_B * lax.rem(blk, 8)
            pltpu.make_async_copy(
                stage.at[f, pl.ds(8 * dg, 8), pl.ds(0, BLK_B)],
                out_hbm.at[f, dg, bg, :, pl.ds(bi0, BLK_B)],
                wsem,
            ).wait()

        def block_body(blk, carry):
            jb = blk * CPB

            gather_start(jb, 0)
            gather_start(jb + 1, 1)

            @pl.when(blk > 0)
            def _():
                for f in range(FIELDS):
                    for dg in range(4):
                        write_wait(f, dg, blk - 1)

            gather_wait(0)
            scatter(0, 0)
            gather_start(jb + 2, 0)
            gather_wait(1)
            scatter(1, 1)
            gather_start(jb + 3, 1)
            gather_wait(0)
            scatter(2, 0)
            gather_wait(1)
            scatter(3, 1)

            for f in range(FIELDS):
                for dg in range(4):
                    write_start(f, dg, blk)

            return carry

        lax.fori_loop(0, NBLK, block_body, 0)
        for f in range(FIELDS):
            for dg in range(4):
                write_wait(f, dg, NBLK - 1)

    return k(tpad, idx_flat)


def kernel(table, indices):
    batch, fields = indices.shape
    tpad = jnp.pad(table, ((0, 0), (0, 128 - DIM)))
    idx_flat = indices.reshape(batch * fields)
    out5d = _gather_sc(tpad, idx_flat)
    return out5d.transpose(2, 4, 0, 1, 3).reshape(batch, fields, DIM)


# speedup vs baseline: 1.7228x; 1.0006x over previous
import functools

import jax
import jax.numpy as jnp
from jax import lax
from jax.experimental import pallas as pl
from jax.experimental.pallas import tpu as pltpu
from jax.experimental.pallas import tpu_sc as plsc

DIM = 32
VOCAB = 1000000
FIELDS = 26
NUM_CORES = 2
NUM_SUBCORES = 16
NUM_WORKERS = NUM_CORES * NUM_SUBCORES
CHUNK = 128
BLK_B = 64
CPB = BLK_B * FIELDS // CHUNK
NBLK = 512 // BLK_B
SPAD = 65


@jax.jit
def _gather_sc(tpad, idx_flat):
    b_per_w = idx_flat.shape[0] // NUM_WORKERS
    mesh = plsc.VectorSubcoreMesh(core_axis_name="c", subcore_axis_name="s")

    @functools.partial(
        pl.kernel,
        out_type=jax.ShapeDtypeStruct((FIELDS, 4, 128, 8, 128), jnp.float32),
        mesh=mesh,
        scratch_types=[
            pltpu.VMEM((13312,), jnp.int32),
            [pltpu.VMEM((CHUNK, 128), jnp.float32) for _ in range(2)],
            pltpu.VMEM((FIELDS, 32, SPAD), jnp.float32),
            [pltpu.SemaphoreType.DMA for _ in range(2)],
            pltpu.SemaphoreType.DMA,
        ],
        compiler_params=pltpu.CompilerParams(
            use_tc_tiling_on_sc=False, needs_layout_passes=False
        ),
    )
    def k(table_hbm, idx_hbm, out_hbm, idx_v, rows, stage, gsems, wsem):
        wid = lax.axis_index("s") * NUM_CORES + lax.axis_index("c")
        base = wid * b_per_w
        pltpu.sync_copy(idx_hbm.at[pl.ds(base, b_per_w)], idx_v)

        iota16 = lax.iota(jnp.int32, 16)

        def gather_start(j, b):
            pltpu.async_copy(
                table_hbm.at[idx_v.at[pl.ds(j * CHUNK, CHUNK)]], rows[b], gsems[b]
            )

        def gather_wait(b):
            pltpu.make_async_copy(
                table_hbm.at[idx_v.at[pl.ds(0, CHUNK)]], rows[b], gsems[b]
            ).wait()

        def scatter(j, b):
            for r in range(CHUNK):
                flat = j * CHUNK + r
                f = lax.rem(flat, FIELDS)
                bi = lax.rem(lax.div(flat, FIELDS), BLK_B)
                f_splat = jnp.full((16,), f, jnp.int32)
                bi_splat = jnp.full((16,), bi, jnp.int32)
                for h in range(2):
                    plsc.store_scatter(
                        stage,
                        [f_splat, iota16 + 16 * h, bi_splat],
                        rows[b][r, pl.ds(16 * h, 16)],
                    )

        def write_start(f, dg, blk):
            bg = 4 * wid + lax.div(blk, 2)
            bi0 = BLK_B * lax.rem(blk, 2)
            pltpu.async_copy(
                stage.at[f, pl.ds(8 * dg, 8), pl.ds(0, BLK_B)],
                out_hbm.at[f, dg, bg, :, pl.ds(bi0, BLK_B)],
                wsem,
            )

        def write_wait(f, dg, blk):
            bg = 4 * wid + lax.div(blk, 2)
            bi0 = BLK_B * lax.rem(blk, 2)
            pltpu.make_async_copy(
                stage.at[f, pl.ds(8 * dg, 8), pl.ds(0, BLK_B)],
                out_hbm.at[f, dg, bg, :, pl.ds(bi0, BLK_B)],
                wsem,
            ).wait()

        def block_body(blk, carry):
            jb = blk * CPB

            gather_start(jb, 0)
            gather_start(jb + 1, 1)

            @pl.when(blk > 0)
            def _():
                for f in range(FIELDS):
                    for dg in range(4):
                        write_wait(f, dg, blk - 1)

            def pair(p, c2):
                j0 = 2 * p
                gather_wait(0)
                scatter(j0, 0)
                gather_start(jb + j0 + 2, 0)
                gather_wait(1)
                scatter(j0 + 1, 1)

                @pl.when(p < (CPB - 3) // 2)
                def _():
                    gather_start(jb + j0 + 3, 1)

                return c2

            lax.fori_loop(0, (CPB - 1) // 2, pair, 0)
            gather_wait(0)
            scatter(CPB - 1, 0)

            for f in range(FIELDS):
                for dg in range(4):
                    write_start(f, dg, blk)

            return carry

        lax.fori_loop(0, NBLK, block_body, 0)
        for f in range(FIELDS):
            for dg in range(4):
                write_wait(f, dg, NBLK - 1)

    return k(tpad, idx_flat)


def kernel(table, indices):
    batch, fields = indices.shape
    tpad = jnp.pad(table, ((0, 0), (0, 128 - DIM)))
    idx_flat = indices.reshape(batch * fields)
    out5d = _gather_sc(tpad, idx_flat)
    return out5d.transpose(2, 4, 0, 1, 3).reshape(batch, fields, DIM)


# speedup vs baseline: 1.7475x; 1.0144x over previous
import functools

import jax
import jax.numpy as jnp
from jax import lax
from jax.experimental import pallas as pl
from jax.experimental.pallas import tpu as pltpu
from jax.experimental.pallas import tpu_sc as plsc

DIM = 32
NUM_CORES = 2
NUM_SUBCORES = 16
NUM_WORKERS = NUM_CORES * NUM_SUBCORES
CHUNK = 128
NBUF = 8


@functools.partial(jax.jit, static_argnames=("b_per_w",))
def _gather_sc(table, idx_flat, b_per_w):
    n_chunks = b_per_w // CHUNK
    n_groups = n_chunks // NBUF
    mesh = plsc.VectorSubcoreMesh(core_axis_name="c", subcore_axis_name="s")

    @functools.partial(
        pl.kernel,
        out_type=jax.ShapeDtypeStruct((idx_flat.shape[0], DIM), jnp.float32),
        mesh=mesh,
        scratch_types=[
            pltpu.VMEM((b_per_w,), jnp.int32),
            [pltpu.VMEM((CHUNK, DIM), jnp.float32) for _ in range(NBUF)],
            [pltpu.SemaphoreType.DMA for _ in range(NBUF)],
            [pltpu.SemaphoreType.DMA for _ in range(NBUF)],
        ],
        compiler_params=pltpu.CompilerParams(use_tc_tiling_on_sc=False),
    )
    def k(table_hbm, idx_hbm, out_hbm, idx_v, rows, gsems, wsems):
        wid = lax.axis_index("s") * NUM_CORES + lax.axis_index("c")
        base = wid * b_per_w
        pltpu.sync_copy(idx_hbm.at[pl.ds(base, b_per_w)], idx_v)

        def gather_start(j, b):
            pltpu.async_copy(
                table_hbm.at[idx_v.at[pl.ds(j * CHUNK, CHUNK)]], rows[b], gsems[b]
            )

        def gather_wait(b):
            pltpu.make_async_copy(
                table_hbm.at[idx_v.at[pl.ds(0, CHUNK)]], rows[b], gsems[b]
            ).wait()

        def write_start(j, b):
            pltpu.async_copy(
                rows[b], out_hbm.at[pl.ds(base + j * CHUNK, CHUNK)], wsems[b]
            )

        def write_wait(j, b):
            pltpu.make_async_copy(
                rows[b], out_hbm.at[pl.ds(base + j * CHUNK, CHUNK)], wsems[b]
            ).wait()

        for b in range(NBUF):
            gather_start(b, b)

        def body(g, carry):
            j0 = g * NBUF
            for b in range(NBUF):
                gather_wait(b)
                write_start(j0 + b, b)
            for b in range(NBUF):
                write_wait(j0 + b, b)
                gather_start(j0 + b + NBUF, b)
            return carry

        lax.fori_loop(0, n_groups - 1, body, 0)
        j0 = (n_groups - 1) * NBUF
        for b in range(NBUF):
            gather_wait(b)
            write_start(j0 + b, b)
        for b in range(NBUF):
            write_wait(j0 + b, b)

    return k(table, idx_flat)


def kernel(table, indices):
    batch, fields = indices.shape
    total = batch * fields
    idx_flat = indices.reshape(total)
    out = _gather_sc(table, idx_flat, total // NUM_WORKERS)
    return out.reshape(batch, fields, DIM)
